# Initial kernel scaffold; baseline (speedup 1.0000x reference)
import jax
import jax.numpy as jnp
from jax.experimental import pallas as pl


def kernel(pc1, pc2):
    raise NotImplementedError("write your pallas kernel here")


# speedup vs baseline: 2.5863x; 2.5863x over previous
import functools
import math

import jax
import jax.numpy as jnp
from jax.experimental import pallas as pl
from jax.experimental.pallas import tpu as pltpu

_EPS = 0.05
_ITERS = 200


def _emd_kernel(pc1_ref, pc2t_ref, out_ref, negc_ref, *, n_pts, n_iters):
    eps = jnp.float32(_EPS)
    log_w = jnp.float32(-math.log(n_pts))

    a = pc1_ref[0]
    bt = pc2t_ref[0]

    d2 = jnp.zeros((n_pts, n_pts), jnp.float32)
    for k in range(3):
        diff = a[:, k : k + 1] - bt[k : k + 1, :]
        d2 = d2 + diff * diff
    negc_ref[...] = -jnp.sqrt(d2 + jnp.float32(1e-12)) / eps

    def body(_, carry):
        f, g = carry
        negc = negc_ref[...]
        m1 = negc + (log_w + g / eps)
        mx1 = jnp.max(m1, axis=1, keepdims=True)
        s1 = jnp.sum(jnp.exp(m1 - mx1), axis=1, keepdims=True)
        f = -eps * (mx1 + jnp.log(s1))
        m2 = negc + (log_w + f / eps)
        mx2 = jnp.max(m2, axis=0, keepdims=True)
        s2 = jnp.sum(jnp.exp(m2 - mx2), axis=0, keepdims=True)
        g = -eps * (mx2 + jnp.log(s2))
        return f, g

    f0 = jnp.zeros((n_pts, 1), jnp.float32)
    g0 = jnp.zeros((1, n_pts), jnp.float32)
    f, g = jax.lax.fori_loop(0, n_iters, body, (f0, g0))

    negc = negc_ref[...]
    logp = 2.0 * log_w + f / eps + g / eps + negc
    total = jnp.sum(jnp.exp(logp) * (-eps * negc))
    out_ref[...] = jnp.full(out_ref.shape, total, jnp.float32)


def kernel(pc1, pc2):
    b, n, _ = pc1.shape
    pc2t = pc2.transpose(0, 2, 1)
    per_batch = pl.pallas_call(
        functools.partial(_emd_kernel, n_pts=n, n_iters=_ITERS),
        grid=(b,),
        in_specs=[
            pl.BlockSpec((1, n, 3), lambda i: (i, 0, 0)),
            pl.BlockSpec((1, 3, n), lambda i: (i, 0, 0)),
        ],
        out_specs=pl.BlockSpec((1, 1, 128), lambda i: (i, 0, 0)),
        out_shape=jax.ShapeDtypeStruct((b, 1, 128), jnp.float32),
        scratch_shapes=[pltpu.VMEM((n, n), jnp.float32)],
        compiler_params=pltpu.CompilerParams(
            dimension_semantics=("parallel",),
            vmem_limit_bytes=100 * 1024 * 1024,
        ),
        name="sinkhorn_emd",
    )(pc1, pc2t)
    return jnp.sum(per_batch[:, 0, 0])


# speedup vs baseline: 4.9848x; 1.9274x over previous
import functools
import math

import jax
import jax.numpy as jnp
from jax.experimental import pallas as pl
from jax.experimental.pallas import tpu as pltpu

_EPS = 0.05
_ITERS = 200


def _emd_kernel(pc1_ref, pc2t_ref, out_ref, negc_ref, *, n_pts, n_iters):
    eps = jnp.float32(_EPS)
    log_w = jnp.float32(-math.log(n_pts))

    a = pc1_ref[0]
    bt = pc2t_ref[0]

    d2 = jnp.zeros((n_pts, n_pts), jnp.float32)
    for k in range(3):
        diff = a[:, k : k + 1] - bt[k : k + 1, :]
        d2 = d2 + diff * diff
    negc_ref[...] = -jnp.sqrt(d2 + jnp.float32(1e-12)) / eps

    tiny = jnp.float32(1e-30)
    ones_mxu = jnp.ones((n_pts, 128), jnp.float32)

    def body(_, carry):
        f, g = carry
        negc = negc_ref[...]
        e1 = jnp.exp(negc + (log_w + g / eps) + f / eps)
        s1 = jnp.maximum(jnp.dot(e1, ones_mxu)[:, :1], tiny)
        s2 = jnp.maximum(
            jnp.sum(e1 * (1.0 / s1), axis=0, keepdims=True), tiny)
        f = f - eps * jnp.log(s1)
        g = g - eps * jnp.log(s2)
        return f, g

    f0 = jnp.zeros((n_pts, 1), jnp.float32)
    g0 = jnp.zeros((1, n_pts), jnp.float32)
    f, g = jax.lax.fori_loop(0, n_iters, body, (f0, g0))

    negc = negc_ref[...]
    logp = 2.0 * log_w + f / eps + g / eps + negc
    total = jnp.sum(jnp.exp(logp) * (-eps * negc))
    out_ref[...] = jnp.full(out_ref.shape, total, jnp.float32)


def kernel(pc1, pc2):
    b, n, _ = pc1.shape
    pc2t = pc2.transpose(0, 2, 1)
    per_batch = pl.pallas_call(
        functools.partial(_emd_kernel, n_pts=n, n_iters=_ITERS),
        grid=(b,),
        in_specs=[
            pl.BlockSpec((1, n, 3), lambda i: (i, 0, 0)),
            pl.BlockSpec((1, 3, n), lambda i: (i, 0, 0)),
        ],
        out_specs=pl.BlockSpec((1, 1, 128), lambda i: (i, 0, 0)),
        out_shape=jax.ShapeDtypeStruct((b, 1, 128), jnp.float32),
        scratch_shapes=[pltpu.VMEM((n, n), jnp.float32)],
        compiler_params=pltpu.CompilerParams(
            dimension_semantics=("parallel",),
            vmem_limit_bytes=100 * 1024 * 1024,
        ),
        name="sinkhorn_emd",
    )(pc1, pc2t)
    return jnp.sum(per_batch[:, 0, 0])
